# baseline (speedup 1.0000x reference)
import functools

import numpy as np
import jax
import jax.numpy as jnp
from jax import lax
from jax.experimental import pallas as pl
from jax.experimental.pallas import tpu as pltpu
from jax.experimental.pallas import tpu_sc as plsc

_MAX_LEN = 2048
_NC = 2
_NS = 16
_NW = _NC * _NS
_LANES = 16


def _pos_table(max_len, E):
    pos = np.arange(max_len, dtype=np.float64)[:, None]
    i = np.arange(E, dtype=np.float64)[None, :]
    angle = pos / np.power(10000.0, (i - np.mod(i, 2)) / E)
    angle[:, 0::2] = np.sin(angle[:, 0::2])
    angle[:, 1::2] = np.cos(angle[:, 1::2])
    return jnp.asarray(angle, dtype=jnp.float32)


def _sc_call(x, pos):
    B, S, E = x.shape
    ROWS = S // _NW
    CH = 8
    NCH = ROWS // CH
    NVEC = E // _LANES
    VEC = CH * NVEC

    mesh = plsc.VectorSubcoreMesh(core_axis_name="c", subcore_axis_name="s")

    @functools.partial(
        pl.kernel,
        mesh=mesh,
        out_type=jax.ShapeDtypeStruct((B, S, E), jnp.float32),
        scratch_types=[
            pltpu.VMEM((ROWS, E), jnp.float32),
            pltpu.VMEM((CH, E), jnp.float32),
            pltpu.VMEM((CH, E), jnp.float32),
        ],
    )
    def k(x_hbm, pos_hbm, out_hbm, pos_v, xin, xout):
        wid = lax.axis_index("s") * _NC + lax.axis_index("c")
        base = wid * ROWS
        pltpu.sync_copy(pos_hbm.at[pl.ds(base, ROWS)], pos_v)
        for b in range(B):
            for kch in range(NCH):
                s0 = kch * CH
                pltpu.sync_copy(x_hbm.at[b, pl.ds(base + s0, CH)], xin)

                def vec_body(i, _, s0=s0):
                    r = i // NVEC
                    c = i - r * NVEC
                    xv = xin[r, pl.ds(c * _LANES, _LANES)]
                    pv = pos_v[s0 + r, pl.ds(c * _LANES, _LANES)]
                    xout[r, pl.ds(c * _LANES, _LANES)] = jnp.where(
                        xv == 0.0, 0.0, pv)
                    return 0

                lax.fori_loop(0, VEC, vec_body, 0)
                pltpu.sync_copy(xout, out_hbm.at[b, pl.ds(base + s0, CH)])

    return k(x, pos)


def kernel(x):
    B, S, E = x.shape
    assert S % _NW == 0 and E % _LANES == 0
    pos = _pos_table(_MAX_LEN, E)[:S]
    return _sc_call(x, pos)


# speedup vs baseline: 1.1065x; 1.1065x over previous
import functools

import numpy as np
import jax
import jax.numpy as jnp
from jax import lax
from jax.experimental import pallas as pl
from jax.experimental.pallas import tpu as pltpu
from jax.experimental.pallas import tpu_sc as plsc

_MAX_LEN = 2048
_NC = 2
_NS = 16
_NW = _NC * _NS
_L = 16


def _pos_table(max_len, E):
    pos = np.arange(max_len, dtype=np.float64)[:, None]
    i = np.arange(E, dtype=np.float64)[None, :]
    angle = pos / np.power(10000.0, (i - np.mod(i, 2)) / E)
    angle[:, 0::2] = np.sin(angle[:, 0::2])
    angle[:, 1::2] = np.cos(angle[:, 1::2])
    return jnp.asarray(angle, dtype=jnp.float32)


def _sc_call(x2, pos1, B, S, E):
    ROWS = S // _NW
    CH = 8
    NCH = ROWS // CH
    CW = CH * E
    UNROLL = 4
    NIT = CW // (_L * UNROLL)

    mesh = plsc.VectorSubcoreMesh(core_axis_name="c", subcore_axis_name="s")

    @functools.partial(
        pl.kernel,
        mesh=mesh,
        out_type=jax.ShapeDtypeStruct((B, S * E), jnp.float32),
        scratch_types=[
            pltpu.VMEM((ROWS * E,), jnp.float32),
            pltpu.VMEM((CW,), jnp.float32),
            pltpu.VMEM((CW,), jnp.float32),
            pltpu.VMEM((CW,), jnp.float32),
            pltpu.VMEM((CW,), jnp.float32),
            pltpu.SemaphoreType.DMA,
            pltpu.SemaphoreType.DMA,
            pltpu.SemaphoreType.DMA,
            pltpu.SemaphoreType.DMA,
            pltpu.SemaphoreType.DMA,
        ],
    )
    def k(x_hbm, pos_hbm, out_hbm, pos_v, xin0, xin1, xout0, xout1,
          sem_pos, sem_i0, sem_i1, sem_o0, sem_o1):
        wid = lax.axis_index("s") * _NC + lax.axis_index("c")
        base = wid * (ROWS * E)

        pos_dma = pltpu.async_copy(
            pos_hbm.at[pl.ds(wid * ROWS * E, ROWS * E)], pos_v, sem_pos)

        xin = (xin0, xin1)
        xout = (xout0, xout1)
        sem_i = (sem_i0, sem_i1)
        sem_o = (sem_o0, sem_o1)

        NT = B * NCH

        def chunk_src(t):
            b, kch = divmod(t, NCH)
            return x_hbm.at[b, pl.ds(base + kch * CW, CW)]

        def chunk_dst(t):
            b, kch = divmod(t, NCH)
            return out_hbm.at[b, pl.ds(base + kch * CW, CW)]

        in_dma = [None] * NT
        out_dma = [None] * NT
        in_dma[0] = pltpu.async_copy(chunk_src(0), xin[0], sem_i[0])
        pos_dma.wait()

        for t in range(NT):
            p = t % 2
            in_dma[t].wait()
            if t + 1 < NT:
                in_dma[t + 1] = pltpu.async_copy(
                    chunk_src(t + 1), xin[1 - p], sem_i[1 - p])
            if t >= 2:
                out_dma[t - 2].wait()
            pbase = (t % NCH) * CW

            def vec_body(i, _, pbase=pbase, xi=xin[p], xo=xout[p]):
                off = i * (_L * UNROLL)
                for u in range(UNROLL):
                    o = off + u * _L
                    xv = xi[pl.ds(o, _L)]
                    pv = pos_v[pl.ds(pbase + o, _L)]
                    xo[pl.ds(o, _L)] = jnp.where(xv == 0.0, 0.0, pv)
                return 0

            lax.fori_loop(0, NIT, vec_body, 0)
            out_dma[t] = pltpu.async_copy(xout[p], chunk_dst(t), sem_o[p])

        out_dma[NT - 2].wait()
        out_dma[NT - 1].wait()

    return k(x2, pos1)


def kernel(x):
    B, S, E = x.shape
    assert S % _NW == 0 and E % _L == 0
    pos = _pos_table(_MAX_LEN, E)[:S]
    x2 = jnp.reshape(x, (B, S * E))
    pos1 = jnp.reshape(pos, (S * E,))
    out2 = _sc_call(x2, pos1, B, S, E)
    return jnp.reshape(out2, (B, S, E))


# speedup vs baseline: 1.1509x; 1.0402x over previous
import functools

import numpy as np
import jax
import jax.numpy as jnp
from jax import lax
from jax.experimental import pallas as pl
from jax.experimental.pallas import tpu as pltpu
from jax.experimental.pallas import tpu_sc as plsc

_MAX_LEN = 2048
_NC = 2
_NS = 16
_NW = _NC * _NS
_L = 16


def _pos_table(max_len, E):
    pos = np.arange(max_len, dtype=np.float64)[:, None]
    i = np.arange(E, dtype=np.float64)[None, :]
    angle = pos / np.power(10000.0, (i - np.mod(i, 2)) / E)
    angle[:, 0::2] = np.sin(angle[:, 0::2])
    angle[:, 1::2] = np.cos(angle[:, 1::2])
    return jnp.asarray(angle, dtype=jnp.float32)


def _sc_call(x2, pos1, B, S, E):
    ROWS = S // _NW
    CH = 8
    NCH = ROWS // CH
    CW = CH * E
    UNROLL = 4
    NIT = CW // (_L * UNROLL)

    mesh = plsc.VectorSubcoreMesh(core_axis_name="c", subcore_axis_name="s")

    @functools.partial(
        pl.kernel,
        mesh=mesh,
        out_type=jax.ShapeDtypeStruct((B, S * E), jnp.float32),
        scratch_types=[
            pltpu.VMEM((ROWS * E,), jnp.float32),
            pltpu.VMEM((CW,), jnp.float32),
            pltpu.VMEM((CW,), jnp.float32),
            pltpu.VMEM((CW,), jnp.float32),
            pltpu.VMEM((CW,), jnp.float32),
            pltpu.SemaphoreType.DMA,
            pltpu.SemaphoreType.DMA,
            pltpu.SemaphoreType.DMA,
            pltpu.SemaphoreType.DMA,
            pltpu.SemaphoreType.DMA,
        ],
    )
    def k(x_hbm, pos_hbm, out_hbm, pos_v, xin0, xin1, xout0, xout1,
          sem_pos, sem_i0, sem_i1, sem_o0, sem_o1):
        wid = lax.axis_index("s") * _NC + lax.axis_index("c")
        base = wid * (ROWS * E)

        pos_dma = pltpu.async_copy(
            pos_hbm.at[pl.ds(wid * ROWS * E, ROWS * E)], pos_v, sem_pos)

        xin = (xin0, xin1)
        xout = (xout0, xout1)
        sem_i = (sem_i0, sem_i1)
        sem_o = (sem_o0, sem_o1)

        NT = B * NCH

        def chunk_src(t):
            b, kch = divmod(t, NCH)
            return x_hbm.at[b, pl.ds(base + kch * CW, CW)]

        def chunk_dst(t):
            b, kch = divmod(t, NCH)
            return out_hbm.at[b, pl.ds(base + kch * CW, CW)]

        in_dma = [None] * NT
        out_dma = [None] * NT
        in_dma[0] = pltpu.async_copy(chunk_src(0), xin[0], sem_i[0])
        pos_dma.wait()

        for t in range(NT):
            p = t % 2
            in_dma[t].wait()
            if t + 1 < NT:
                in_dma[t + 1] = pltpu.async_copy(
                    chunk_src(t + 1), xin[1 - p], sem_i[1 - p])
            if t >= 2:
                out_dma[t - 2].wait()
            pbase = (t % NCH) * CW
            xi, xo = xin[p], xout[p]

            @plsc.parallel_loop(0, CW, step=_L, unroll=UNROLL)
            def vec_body(i, pbase=pbase, xi=xi, xo=xo):
                xv = xi[pl.ds(i, _L)]
                pv = pos_v[pl.ds(pbase + i, _L)]
                xo[pl.ds(i, _L)] = jnp.where(xv == 0.0, 0.0, pv)
            out_dma[t] = pltpu.async_copy(xout[p], chunk_dst(t), sem_o[p])

        out_dma[NT - 2].wait()
        out_dma[NT - 1].wait()

    return k(x2, pos1)


def kernel(x):
    B, S, E = x.shape
    assert S % _NW == 0 and E % _L == 0
    pos = _pos_table(_MAX_LEN, E)[:S]
    x2 = jnp.reshape(x, (B, S * E))
    pos1 = jnp.reshape(pos, (S * E,))
    out2 = _sc_call(x2, pos1, B, S, E)
    return jnp.reshape(out2, (B, S, E))


# speedup vs baseline: 1.8966x; 1.6479x over previous
import functools

import numpy as np
import jax
import jax.numpy as jnp
from jax import lax
from jax.experimental import pallas as pl
from jax.experimental.pallas import tpu as pltpu
from jax.experimental.pallas import tpu_sc as plsc

_MAX_LEN = 2048
_NC = 2
_NS = 16
_NW = _NC * _NS
_L = 16


def _pos_table(max_len, E):
    pos = np.arange(max_len, dtype=np.float64)[:, None]
    i = np.arange(E, dtype=np.float64)[None, :]
    angle = pos / np.power(10000.0, (i - np.mod(i, 2)) / E)
    angle[:, 0::2] = np.sin(angle[:, 0::2])
    angle[:, 1::2] = np.cos(angle[:, 1::2])
    return jnp.asarray(angle, dtype=jnp.float32)


def _sc_call(x, pos):
    B, S, E = x.shape
    ROWS = S // _NW
    CH = 8
    NCH = ROWS // CH
    UNROLL = 2

    mesh = plsc.VectorSubcoreMesh(core_axis_name="c", subcore_axis_name="s")

    @functools.partial(
        pl.kernel,
        mesh=mesh,
        out_type=jax.ShapeDtypeStruct((B, S, E), jnp.float32),
        scratch_types=[
            pltpu.VMEM((ROWS, E), jnp.float32),
            pltpu.VMEM((CH, E), jnp.float32),
            pltpu.VMEM((CH, E), jnp.float32),
            pltpu.VMEM((CH, E), jnp.float32),
            pltpu.VMEM((CH, E), jnp.float32),
            pltpu.SemaphoreType.DMA,
            pltpu.SemaphoreType.DMA,
            pltpu.SemaphoreType.DMA,
            pltpu.SemaphoreType.DMA,
            pltpu.SemaphoreType.DMA,
        ],
    )
    def k(x_hbm, pos_hbm, out_hbm, pos_v, xin0, xin1, xout0, xout1,
          sem_pos, sem_i0, sem_i1, sem_o0, sem_o1):
        wid = lax.axis_index("s") * _NC + lax.axis_index("c")
        base = wid * ROWS

        pos_dma = pltpu.async_copy(
            pos_hbm.at[pl.ds(base, ROWS), :], pos_v, sem_pos)

        xin = (xin0, xin1)
        xout = (xout0, xout1)
        sem_i = (sem_i0, sem_i1)
        sem_o = (sem_o0, sem_o1)

        NT = B * NCH

        def chunk_ref(hbm, t):
            b, kch = divmod(t, NCH)
            return hbm.at[b, pl.ds(base + kch * CH, CH), :]

        in_dma = [None] * NT
        out_dma = [None] * NT
        in_dma[0] = pltpu.async_copy(chunk_ref(x_hbm, 0), xin[0], sem_i[0])
        pos_dma.wait()

        for t in range(NT):
            p = t % 2
            in_dma[t].wait()
            if t + 1 < NT:
                in_dma[t + 1] = pltpu.async_copy(
                    chunk_ref(x_hbm, t + 1), xin[1 - p], sem_i[1 - p])
            if t >= 2:
                out_dma[t - 2].wait()
            r0 = (t % NCH) * CH
            xi, xo = xin[p], xout[p]

            @plsc.parallel_loop(0, E, step=_L, unroll=UNROLL)
            def vec_body(i, r0=r0, xi=xi, xo=xo):
                for r in range(CH):
                    xv = xi[r, pl.ds(i, _L)]
                    pv = pos_v[r0 + r, pl.ds(i, _L)]
                    xo[r, pl.ds(i, _L)] = jnp.where(xv == 0.0, 0.0, pv)

            out_dma[t] = pltpu.async_copy(xo, chunk_ref(out_hbm, t), sem_o[p])

        out_dma[NT - 2].wait()
        out_dma[NT - 1].wait()

    return k(x, pos)


def kernel(x):
    B, S, E = x.shape
    assert S % _NW == 0 and E % _L == 0
    pos = _pos_table(_MAX_LEN, E)[:S]
    return _sc_call(x, pos)
